# baseline (speedup 1.0000x reference)
import jax
import jax.numpy as jnp
from jax import lax
from jax.experimental import pallas as pl
from jax.experimental.pallas import tpu as pltpu

N_WAY = 64
K_SHOT = 5
Q_QUERY = 20
D_IN = 2048
D_OUT = 1024
NQ = N_WAY * Q_QUERY
TOPK = 10
NEG_INF = -1e30
BLK = 256

_F32 = jnp.float32
_BF16 = jnp.bfloat16


def _feat_body(x_ref, w_ref, b_ref, o_ref):
    o_ref[...] = (
        jnp.dot(x_ref[...], w_ref[...], preferred_element_type=_F32) + b_ref[...]
    )


def _encode(xb, Wb, b):
    rows = N_WAY * (K_SHOT + Q_QUERY)
    blk = 200
    return pl.pallas_call(
        _feat_body,
        grid=(rows // blk,),
        in_specs=[
            pl.BlockSpec((blk, D_IN), lambda i: (i, 0)),
            pl.BlockSpec((D_IN, D_OUT), lambda i: (0, 0)),
            pl.BlockSpec((1, D_OUT), lambda i: (0, 0)),
        ],
        out_specs=pl.BlockSpec((blk, D_OUT), lambda i: (i, 0)),
        out_shape=jax.ShapeDtypeStruct((rows, D_OUT), _F32),
    )(xb, Wb, b.reshape(1, D_OUT))


def _prep_body(sup_ref, qry_ref, proto_ref, ap_ref, qn_ref):
    query = qry_ref[...]
    query_b = query.astype(_BF16)
    proto = sup_ref[:, 0 * D_OUT : 1 * D_OUT]
    for s in range(1, K_SHOT):
        proto = proto + sup_ref[:, s * D_OUT : (s + 1) * D_OUT]
    proto = proto * (1.0 / K_SHOT)
    proto_ref[...] = proto

    qn1 = jnp.sum(query * query, axis=1)
    qn_ref[...] = qn1[:, None]
    pn = jnp.sum(proto * proto, axis=1)

    pq = lax.dot_general(
        proto.astype(_BF16), query_b, (((1,), (1,)), ((), ())),
        preferred_element_type=_F32,
    )
    ps_t = 2.0 * pq - pn[:, None] - qn1[None, :]

    row_n = lax.broadcasted_iota(jnp.int32, (N_WAY, NQ), 0).astype(_F32)
    m_col = jnp.max(ps_t, axis=0, keepdims=True)
    label = jnp.min(
        jnp.where(ps_t == m_col, row_n, jnp.float32(N_WAY)), axis=0, keepdims=True
    )
    assign = row_n == label

    logq = jnp.where(assign, ps_t, NEG_INF)
    mx = jnp.maximum(jnp.max(logq, axis=1, keepdims=True), 0.0)
    e = jnp.exp(logq - mx)
    e_self = jnp.exp(-mx)
    den = jnp.sum(e, axis=1, keepdims=True) + e_self
    wq = e / den
    ap_ref[...] = (
        lax.dot_general(wq.astype(_BF16), query_b, (((1,), (0,)), ((), ())),
                        preferred_element_type=_F32)
        + (e_self / den) * proto
    )


def _prep(sup, qry):
    return pl.pallas_call(
        _prep_body,
        in_specs=[
            pl.BlockSpec(memory_space=pltpu.VMEM),
            pl.BlockSpec(memory_space=pltpu.VMEM),
        ],
        out_specs=[
            pl.BlockSpec(memory_space=pltpu.VMEM),
            pl.BlockSpec(memory_space=pltpu.VMEM),
            pl.BlockSpec(memory_space=pltpu.VMEM),
        ],
        out_shape=[
            jax.ShapeDtypeStruct((N_WAY, D_OUT), _F32),
            jax.ShapeDtypeStruct((N_WAY, D_OUT), _F32),
            jax.ShapeDtypeStruct((NQ, 1), _F32),
        ],
    )(sup, qry)


def _knn_body(qblk_ref, qall_ref, qnb_ref, qnt_ref, qsim_ref, idx_ref):
    qq = lax.dot_general(
        qblk_ref[...], qall_ref[...], (((1,), (1,)), ((), ())),
        preferred_element_type=_F32,
    )
    qsim = 2.0 * qq - qnb_ref[...] - qnt_ref[...]
    qsim_ref[...] = qsim

    col = lax.broadcasted_iota(jnp.int32, (BLK, NQ), 1).astype(_F32)
    idx_ref[...] = jnp.full((BLK, 16), float(NQ), dtype=_F32)
    work = qsim
    for t in range(TOPK):
        mt = jnp.max(work, axis=1, keepdims=True)
        sel = jnp.min(
            jnp.where(work == mt, col, jnp.float32(NQ)), axis=1, keepdims=True
        )
        idx_ref[:, t : t + 1] = sel
        hit = col == sel
        work = jnp.where(hit, NEG_INF, work)


def _knn(qry, qn, qnt):
    return pl.pallas_call(
        _knn_body,
        grid=(NQ // BLK,),
        in_specs=[
            pl.BlockSpec((BLK, D_OUT), lambda i: (i, 0)),
            pl.BlockSpec((NQ, D_OUT), lambda i: (0, 0)),
            pl.BlockSpec((BLK, 1), lambda i: (i, 0)),
            pl.BlockSpec((1, NQ), lambda i: (0, 0)),
        ],
        out_specs=[
            pl.BlockSpec((BLK, NQ), lambda i: (i, 0)),
            pl.BlockSpec((BLK, 16), lambda i: (i, 0)),
        ],
        out_shape=[
            jax.ShapeDtypeStruct((NQ, NQ), _F32),
            jax.ShapeDtypeStruct((NQ, 16), _F32),
        ],
    )(qry, qry, qn, qnt)


def _comb_body(qsim_ref, idx_ref, idxt_ref, qall_ref, ap_ref, s_ref, o_ref):
    pid = pl.program_id(0)
    qsim = qsim_ref[...]
    col = lax.broadcasted_iota(jnp.int32, (BLK, NQ), 1).astype(_F32)

    m_blk = jnp.zeros((BLK, NQ), dtype=jnp.bool_)
    mt_blk = jnp.zeros((BLK, NQ), dtype=jnp.bool_)
    row_glob = lax.broadcasted_iota(jnp.int32, (BLK, NQ), 0).astype(_F32) + jnp.float32(BLK) * pid.astype(_F32)
    for t in range(TOPK):
        m_blk = m_blk | (col == idx_ref[:, t : t + 1])
        mt_blk = mt_blk | (idxt_ref[t : t + 1, :] == row_glob)
    mutual = m_blk & mt_blk

    q_log = jnp.where(mutual, qsim, NEG_INF)
    mq = jnp.max(q_log, axis=1, keepdims=True)
    e = jnp.exp(q_log - mq)
    q_w = e / jnp.sum(e, axis=1, keepdims=True)

    aq = lax.dot_general(
        q_w.astype(_BF16), qall_ref[...], (((1,), (0,)), ((), ())),
        preferred_element_type=_F32,
    )

    ap = ap_ref[...]
    apn = jnp.sum(ap * ap, axis=1)
    aqn = jnp.sum(aq * aq, axis=1, keepdims=True)
    aqp = lax.dot_general(
        aq.astype(_BF16), ap.astype(_BF16), (((1,), (1,)), ((), ())),
        preferred_element_type=_F32,
    )
    sim = 2.0 * aqp - aqn - apn[None, :]
    o_ref[...] = s_ref[0] * sim + s_ref[1]


def _combine(qsim, idx, idxt, qry, ap, scal):
    return pl.pallas_call(
        _comb_body,
        grid=(NQ // BLK,),
        in_specs=[
            pl.BlockSpec((BLK, NQ), lambda i: (i, 0)),
            pl.BlockSpec((BLK, 16), lambda i: (i, 0)),
            pl.BlockSpec((16, NQ), lambda i: (0, 0)),
            pl.BlockSpec((NQ, D_OUT), lambda i: (0, 0)),
            pl.BlockSpec((N_WAY, D_OUT), lambda i: (0, 0)),
            pl.BlockSpec(memory_space=pltpu.SMEM),
        ],
        out_specs=pl.BlockSpec((BLK, N_WAY), lambda i: (i, 0)),
        out_shape=jax.ShapeDtypeStruct((NQ, N_WAY), _F32),
    )(qsim, idx, idxt, qry, ap, scal)


def kernel(x, W, b, tao, n, k, q):
    residual = (
        (jnp.asarray(n) - N_WAY)
        + (jnp.asarray(k) - K_SHOT)
        + (jnp.asarray(q) - Q_QUERY)
    ).astype(x.dtype)
    feat = _encode(x.astype(_BF16), W.astype(_BF16), b)
    fr = feat.reshape(N_WAY, K_SHOT + Q_QUERY, D_OUT)
    sup = fr[:, :K_SHOT, :].reshape(N_WAY, K_SHOT * D_OUT)
    qry = fr[:, K_SHOT:, :].reshape(NQ, D_OUT)

    proto, ap, qn = _prep(sup, qry)
    del proto
    qb = qry.astype(_BF16)
    qnt = qn.reshape(1, NQ)
    qsim, idx = _knn(qb, qn, qnt)
    idxt = idx.T
    scal = jnp.stack([tao.astype(_F32), residual.astype(_F32)])
    return _combine(qsim, idx, idxt, qb, ap, scal)


# speedup vs baseline: 1.4993x; 1.4993x over previous
import jax
import jax.numpy as jnp
from jax import lax
from jax.experimental import pallas as pl
from jax.experimental.pallas import tpu as pltpu

N_WAY = 64
K_SHOT = 5
Q_QUERY = 20
D_IN = 2048
D_OUT = 1024
NQ = N_WAY * Q_QUERY
TOPK = 10
NEG_INF = -1e30
BLK = 256

_F32 = jnp.float32
_BF16 = jnp.bfloat16


_CPB = 8


def _feat_body(x_ref, w_ref, b_ref, qry_ref, sup_ref):
    o = jnp.dot(x_ref[...], w_ref[...], preferred_element_type=_F32) + b_ref[...]
    for kk in range(_CPB):
        qry_ref[kk * Q_QUERY : (kk + 1) * Q_QUERY, :] = o[
            kk * 25 + K_SHOT : (kk + 1) * 25, :
        ]
        for s in range(K_SHOT):
            sup_ref[kk : kk + 1, s * D_OUT : (s + 1) * D_OUT] = o[
                kk * 25 + s : kk * 25 + s + 1, :
            ]


def _encode(x, W, b):
    blk = 25 * _CPB
    return pl.pallas_call(
        _feat_body,
        grid=(N_WAY // _CPB,),
        in_specs=[
            pl.BlockSpec((blk, D_IN), lambda i: (i, 0)),
            pl.BlockSpec((D_IN, D_OUT), lambda i: (0, 0)),
            pl.BlockSpec((1, D_OUT), lambda i: (0, 0)),
        ],
        out_specs=[
            pl.BlockSpec((_CPB * Q_QUERY, D_OUT), lambda i: (i, 0)),
            pl.BlockSpec((_CPB, K_SHOT * D_OUT), lambda i: (i, 0)),
        ],
        out_shape=[
            jax.ShapeDtypeStruct((NQ, D_OUT), _F32),
            jax.ShapeDtypeStruct((N_WAY, K_SHOT * D_OUT), _F32),
        ],
    )(x, W, b.reshape(1, D_OUT))


def _prep_body(sup_ref, qry_ref, proto_ref, ap_ref, qn_ref):
    query = qry_ref[...]
    proto = sup_ref[:, 0 * D_OUT : 1 * D_OUT]
    for s in range(1, K_SHOT):
        proto = proto + sup_ref[:, s * D_OUT : (s + 1) * D_OUT]
    proto = proto * (1.0 / K_SHOT)
    proto_ref[...] = proto

    qn1 = jnp.sum(query * query, axis=1)
    qn_ref[...] = qn1[:, None]
    pn = jnp.sum(proto * proto, axis=1)

    pq = lax.dot_general(
        proto, query, (((1,), (1,)), ((), ())), preferred_element_type=_F32
    )
    ps_t = 2.0 * pq - pn[:, None] - qn1[None, :]

    row_n = lax.broadcasted_iota(jnp.int32, (N_WAY, NQ), 0).astype(_F32)
    m_col = jnp.max(ps_t, axis=0, keepdims=True)
    label = jnp.min(
        jnp.where(ps_t == m_col, row_n, jnp.float32(N_WAY)), axis=0, keepdims=True
    )
    assign = row_n == label

    logq = jnp.where(assign, ps_t, NEG_INF)
    mx = jnp.maximum(jnp.max(logq, axis=1, keepdims=True), 0.0)
    e = jnp.exp(logq - mx)
    e_self = jnp.exp(-mx)
    den = jnp.sum(e, axis=1, keepdims=True) + e_self
    wq = e / den
    ap_ref[...] = (
        lax.dot_general(wq, query, (((1,), (0,)), ((), ())), preferred_element_type=_F32)
        + (e_self / den) * proto
    )


def _prep(sup, qry):
    return pl.pallas_call(
        _prep_body,
        in_specs=[
            pl.BlockSpec(memory_space=pltpu.VMEM),
            pl.BlockSpec(memory_space=pltpu.VMEM),
        ],
        out_specs=[
            pl.BlockSpec(memory_space=pltpu.VMEM),
            pl.BlockSpec(memory_space=pltpu.VMEM),
            pl.BlockSpec(memory_space=pltpu.VMEM),
        ],
        out_shape=[
            jax.ShapeDtypeStruct((N_WAY, D_OUT), _F32),
            jax.ShapeDtypeStruct((N_WAY, D_OUT), _F32),
            jax.ShapeDtypeStruct((NQ, 1), _F32),
        ],
    )(sup, qry)


def _knn_body(qblk_ref, qall_ref, qnb_ref, qnt_ref, qsim_ref, idx_ref):
    qq = lax.dot_general(
        qblk_ref[...], qall_ref[...], (((1,), (1,)), ((), ())),
        preferred_element_type=_F32,
    )
    qsim = 2.0 * qq - qnb_ref[...] - qnt_ref[...]
    qsim_ref[...] = qsim

    col = lax.broadcasted_iota(jnp.int32, (BLK, NQ), 1).astype(_F32)
    idx_ref[...] = jnp.full((BLK, 16), float(NQ), dtype=_F32)
    work = qsim
    for t in range(TOPK):
        mt = jnp.max(work, axis=1, keepdims=True)
        sel = jnp.min(
            jnp.where(work == mt, col, jnp.float32(NQ)), axis=1, keepdims=True
        )
        idx_ref[:, t : t + 1] = sel
        hit = col == sel
        work = jnp.where(hit, NEG_INF, work)


def _knn(qry, qn, qnt):
    return pl.pallas_call(
        _knn_body,
        grid=(NQ // BLK,),
        in_specs=[
            pl.BlockSpec((BLK, D_OUT), lambda i: (i, 0)),
            pl.BlockSpec((NQ, D_OUT), lambda i: (0, 0)),
            pl.BlockSpec((BLK, 1), lambda i: (i, 0)),
            pl.BlockSpec((1, NQ), lambda i: (0, 0)),
        ],
        out_specs=[
            pl.BlockSpec((BLK, NQ), lambda i: (i, 0)),
            pl.BlockSpec((BLK, 16), lambda i: (i, 0)),
        ],
        out_shape=[
            jax.ShapeDtypeStruct((NQ, NQ), _F32),
            jax.ShapeDtypeStruct((NQ, 16), _F32),
        ],
    )(qry, qry, qn, qnt)


def _comb_body(qsim_ref, idx_ref, idxt_ref, qall_ref, ap_ref, s_ref, o_ref):
    pid = pl.program_id(0)
    qsim = qsim_ref[...]
    col = lax.broadcasted_iota(jnp.int32, (BLK, NQ), 1).astype(_F32)

    m_blk = jnp.zeros((BLK, NQ), dtype=jnp.bool_)
    mt_blk = jnp.zeros((BLK, NQ), dtype=jnp.bool_)
    row_glob = lax.broadcasted_iota(jnp.int32, (BLK, NQ), 0).astype(_F32) + jnp.float32(BLK) * pid.astype(_F32)
    for t in range(TOPK):
        m_blk = m_blk | (col == idx_ref[:, t : t + 1])
        mt_blk = mt_blk | (idxt_ref[t : t + 1, :] == row_glob)
    mutual = m_blk & mt_blk

    q_log = jnp.where(mutual, qsim, NEG_INF)
    mq = jnp.max(q_log, axis=1, keepdims=True)
    e = jnp.exp(q_log - mq)
    q_w = e / jnp.sum(e, axis=1, keepdims=True)

    aq = lax.dot_general(
        q_w, qall_ref[...], (((1,), (0,)), ((), ())), preferred_element_type=_F32
    )

    ap = ap_ref[...]
    apn = jnp.sum(ap * ap, axis=1)
    aqn = jnp.sum(aq * aq, axis=1, keepdims=True)
    aqp = lax.dot_general(
        aq, ap, (((1,), (1,)), ((), ())), preferred_element_type=_F32
    )
    sim = 2.0 * aqp - aqn - apn[None, :]
    o_ref[...] = s_ref[0] * sim + s_ref[1]


def _combine(qsim, idx, idxt, qry, ap, scal):
    return pl.pallas_call(
        _comb_body,
        grid=(NQ // BLK,),
        in_specs=[
            pl.BlockSpec((BLK, NQ), lambda i: (i, 0)),
            pl.BlockSpec((BLK, 16), lambda i: (i, 0)),
            pl.BlockSpec((16, NQ), lambda i: (0, 0)),
            pl.BlockSpec((NQ, D_OUT), lambda i: (0, 0)),
            pl.BlockSpec((N_WAY, D_OUT), lambda i: (0, 0)),
            pl.BlockSpec(memory_space=pltpu.SMEM),
        ],
        out_specs=pl.BlockSpec((BLK, N_WAY), lambda i: (i, 0)),
        out_shape=jax.ShapeDtypeStruct((NQ, N_WAY), _F32),
    )(qsim, idx, idxt, qry, ap, scal)


def kernel(x, W, b, tao, n, k, q):
    residual = (
        (jnp.asarray(n) - N_WAY)
        + (jnp.asarray(k) - K_SHOT)
        + (jnp.asarray(q) - Q_QUERY)
    ).astype(x.dtype)
    qry, sup = _encode(x, W, b)

    proto, ap, qn = _prep(sup, qry)
    del proto
    qnt = qn.reshape(1, NQ)
    qsim, idx = _knn(qry, qn, qnt)
    idxt = idx.T
    scal = jnp.stack([tao.astype(_F32), residual.astype(_F32)])
    return _combine(qsim, idx, idxt, qry, ap, scal)


# speedup vs baseline: 1.5783x; 1.0527x over previous
import jax
import jax.numpy as jnp
from jax import lax
from jax.experimental import pallas as pl
from jax.experimental.pallas import tpu as pltpu

N_WAY = 64
K_SHOT = 5
Q_QUERY = 20
D_IN = 2048
D_OUT = 1024
NQ = N_WAY * Q_QUERY
TOPK = 10
NEG_INF = -1e30
BLK = 256

_F32 = jnp.float32


_CPB = 8


def _feat_body(x_ref, w_ref, b_ref, qry_ref, sup_ref):
    o = jnp.dot(x_ref[...], w_ref[...], preferred_element_type=_F32) + b_ref[...]
    for kk in range(_CPB):
        qry_ref[kk * Q_QUERY : (kk + 1) * Q_QUERY, :] = o[
            kk * 25 + K_SHOT : (kk + 1) * 25, :
        ]
        for s in range(K_SHOT):
            sup_ref[kk : kk + 1, s * D_OUT : (s + 1) * D_OUT] = o[
                kk * 25 + s : kk * 25 + s + 1, :
            ]


def _encode(x, W, b):
    blk = 25 * _CPB
    return pl.pallas_call(
        _feat_body,
        grid=(N_WAY // _CPB,),
        in_specs=[
            pl.BlockSpec((blk, D_IN), lambda i: (i, 0)),
            pl.BlockSpec((D_IN, D_OUT), lambda i: (0, 0)),
            pl.BlockSpec((1, D_OUT), lambda i: (0, 0)),
        ],
        out_specs=[
            pl.BlockSpec((_CPB * Q_QUERY, D_OUT), lambda i: (i, 0)),
            pl.BlockSpec((_CPB, K_SHOT * D_OUT), lambda i: (i, 0)),
        ],
        out_shape=[
            jax.ShapeDtypeStruct((NQ, D_OUT), _F32),
            jax.ShapeDtypeStruct((N_WAY, K_SHOT * D_OUT), _F32),
        ],
    )(x, W, b.reshape(1, D_OUT))


def _knn_body(qblk_ref, qall_ref, sup_ref, qsim_ref, idx_ref, idxt_ref, ap_ref):
    pid = pl.program_id(0)
    qall = qall_ref[...]
    qnt = lax.dot_general(
        jnp.ones((1, D_OUT), _F32), qall * qall, (((1,), (1,)), ((), ())),
        preferred_element_type=_F32,
    )

    qblk = qblk_ref[...]
    qnb = jnp.sum(qblk * qblk, axis=1, keepdims=True)
    qq = lax.dot_general(
        qblk, qall, (((1,), (1,)), ((), ())), preferred_element_type=_F32
    )
    qsim = 2.0 * qq - qnb - qnt
    qsim_ref[...] = qsim

    col = lax.broadcasted_iota(jnp.int32, (BLK, NQ), 1).astype(_F32)
    col16 = lax.broadcasted_iota(jnp.int32, (BLK, 16), 1).astype(_F32)
    idx_mat = jnp.full((BLK, 16), float(NQ), dtype=_F32)
    work = qsim
    for t in range(TOPK):
        mt = jnp.max(work, axis=1, keepdims=True)
        sel = jnp.min(
            jnp.where(work == mt, col, jnp.float32(NQ)), axis=1, keepdims=True
        )
        idx_mat = jnp.where(col16 == float(t), sel, idx_mat)
        work = jnp.where(col == sel, NEG_INF, work)
    idx_ref[...] = idx_mat
    idxt_ref[...] = idx_mat.T

    @pl.when(pid == 0)
    def _adapted_proto():
        proto = sup_ref[:, 0 * D_OUT : 1 * D_OUT]
        for s in range(1, K_SHOT):
            proto = proto + sup_ref[:, s * D_OUT : (s + 1) * D_OUT]
        proto = proto * (1.0 / K_SHOT)
        pn = jnp.sum(proto * proto, axis=1)
        pq = lax.dot_general(
            proto, qall, (((1,), (1,)), ((), ())), preferred_element_type=_F32
        )
        ps_t = 2.0 * pq - pn[:, None] - qnt

        row_n = lax.broadcasted_iota(jnp.int32, (N_WAY, NQ), 0).astype(_F32)
        m_col = jnp.max(ps_t, axis=0, keepdims=True)
        label = jnp.min(
            jnp.where(ps_t == m_col, row_n, jnp.float32(N_WAY)), axis=0,
            keepdims=True,
        )
        assign = row_n == label

        logq = jnp.where(assign, ps_t, NEG_INF)
        mx = jnp.maximum(jnp.max(logq, axis=1, keepdims=True), 0.0)
        e = jnp.exp(logq - mx)
        e_self = jnp.exp(-mx)
        den = jnp.sum(e, axis=1, keepdims=True) + e_self
        wq = e / den
        ap_ref[...] = (
            lax.dot_general(
                wq, qall, (((1,), (0,)), ((), ())), preferred_element_type=_F32
            )
            + (e_self / den) * proto
        )


def _knn(qry, sup):
    return pl.pallas_call(
        _knn_body,
        grid=(NQ // BLK,),
        in_specs=[
            pl.BlockSpec((BLK, D_OUT), lambda i: (i, 0)),
            pl.BlockSpec((NQ, D_OUT), lambda i: (0, 0)),
            pl.BlockSpec((N_WAY, K_SHOT * D_OUT), lambda i: (0, 0)),
        ],
        out_specs=[
            pl.BlockSpec((BLK, NQ), lambda i: (i, 0)),
            pl.BlockSpec((BLK, 16), lambda i: (i, 0)),
            pl.BlockSpec((16, BLK), lambda i: (0, i)),
            pl.BlockSpec((N_WAY, D_OUT), lambda i: (0, 0)),
        ],
        out_shape=[
            jax.ShapeDtypeStruct((NQ, NQ), _F32),
            jax.ShapeDtypeStruct((NQ, 16), _F32),
            jax.ShapeDtypeStruct((16, NQ), _F32),
            jax.ShapeDtypeStruct((N_WAY, D_OUT), _F32),
        ],
    )(qry, qry, sup)


def _comb_body(qsim_ref, idx_ref, idxt_ref, qall_ref, ap_ref, s_ref, o_ref):
    pid = pl.program_id(0)
    qsim = qsim_ref[...]
    col = lax.broadcasted_iota(jnp.int32, (BLK, NQ), 1).astype(_F32)

    m_blk = jnp.zeros((BLK, NQ), dtype=jnp.bool_)
    mt_blk = jnp.zeros((BLK, NQ), dtype=jnp.bool_)
    row_glob = lax.broadcasted_iota(jnp.int32, (BLK, NQ), 0).astype(
        _F32
    ) + jnp.float32(BLK) * pid.astype(_F32)
    for t in range(TOPK):
        m_blk = m_blk | (col == idx_ref[:, t : t + 1])
        mt_blk = mt_blk | (idxt_ref[t : t + 1, :] == row_glob)
    mutual = m_blk & mt_blk

    q_log = jnp.where(mutual, qsim, NEG_INF)
    mq = jnp.max(q_log, axis=1, keepdims=True)
    e = jnp.exp(q_log - mq)
    q_w = e / jnp.sum(e, axis=1, keepdims=True)

    aq = lax.dot_general(
        q_w, qall_ref[...], (((1,), (0,)), ((), ())), preferred_element_type=_F32
    )

    ap = ap_ref[...]
    apn = jnp.sum(ap * ap, axis=1)
    aqn = jnp.sum(aq * aq, axis=1, keepdims=True)
    aqp = lax.dot_general(
        aq, ap, (((1,), (1,)), ((), ())), preferred_element_type=_F32
    )
    sim = 2.0 * aqp - aqn - apn[None, :]
    o_ref[...] = s_ref[0] * sim + s_ref[1]


def _combine(qsim, idx, idxt, qry, ap, scal):
    return pl.pallas_call(
        _comb_body,
        grid=(NQ // BLK,),
        in_specs=[
            pl.BlockSpec((BLK, NQ), lambda i: (i, 0)),
            pl.BlockSpec((BLK, 16), lambda i: (i, 0)),
            pl.BlockSpec((16, NQ), lambda i: (0, 0)),
            pl.BlockSpec((NQ, D_OUT), lambda i: (0, 0)),
            pl.BlockSpec((N_WAY, D_OUT), lambda i: (0, 0)),
            pl.BlockSpec(memory_space=pltpu.SMEM),
        ],
        out_specs=pl.BlockSpec((BLK, N_WAY), lambda i: (i, 0)),
        out_shape=jax.ShapeDtypeStruct((NQ, N_WAY), _F32),
    )(qsim, idx, idxt, qry, ap, scal)


def kernel(x, W, b, tao, n, k, q):
    residual = (
        (jnp.asarray(n) - N_WAY)
        + (jnp.asarray(k) - K_SHOT)
        + (jnp.asarray(q) - Q_QUERY)
    ).astype(x.dtype)
    qry, sup = _encode(x, W, b)
    qsim, idx, idxt, ap = _knn(qry, sup)
    scal = jnp.stack([tao.astype(_F32), residual.astype(_F32)])
    return _combine(qsim, idx, idxt, qry, ap, scal)


# speedup vs baseline: 1.8001x; 1.1405x over previous
import jax
import jax.numpy as jnp
from jax import lax
from jax.experimental import pallas as pl
from jax.experimental.pallas import tpu as pltpu

N_WAY = 64
K_SHOT = 5
Q_QUERY = 20
D_IN = 2048
D_OUT = 1024
NQ = N_WAY * Q_QUERY
TOPK = 10
NEG_INF = -1e30
BLK = 256
_CPB = 8
_NE = N_WAY // _CPB
_NB = NQ // BLK

_F32 = jnp.float32


def _body(x_ref, w_ref, b_ref, s_ref, o_ref,
          qry_s, sup_s, qsim_s, idx_s, idxt_s, ap_s):
    pid = pl.program_id(0)

    @pl.when(pid < _NE)
    def _encode():
        o = jnp.dot(x_ref[...], w_ref[...], preferred_element_type=_F32) + b_ref[...]
        base_q = pl.multiple_of(pid * (_CPB * Q_QUERY), _CPB * Q_QUERY)
        base_c = pl.multiple_of(pid * _CPB, _CPB)
        qry_s[pl.ds(base_q, _CPB * Q_QUERY), :] = jnp.concatenate(
            [o[kk * 25 + K_SHOT : (kk + 1) * 25, :] for kk in range(_CPB)], axis=0
        )
        sup_s[pl.ds(base_c, _CPB), :] = jnp.concatenate(
            [
                jnp.concatenate(
                    [o[kk * 25 + s : kk * 25 + s + 1, :] for s in range(K_SHOT)],
                    axis=1,
                )
                for kk in range(_CPB)
            ],
            axis=0,
        )

    @pl.when((pid >= _NE) & (pid < _NE + _NB))
    def _knn():
        j = pid - _NE
        qall = qry_s[...]
        qnt = lax.dot_general(
            jnp.ones((1, D_OUT), _F32), qall * qall, (((1,), (1,)), ((), ())),
            preferred_element_type=_F32,
        )

        qblk = qry_s[pl.ds(pl.multiple_of(j * BLK, BLK), BLK), :]
        qnb = jnp.sum(qblk * qblk, axis=1, keepdims=True)
        qq = lax.dot_general(
            qblk, qall, (((1,), (1,)), ((), ())), preferred_element_type=_F32
        )
        qsim = 2.0 * qq - qnb - qnt
        qsim_s[pl.ds(pl.multiple_of(j * BLK, BLK), BLK), :] = qsim

        col = lax.broadcasted_iota(jnp.int32, (BLK, NQ), 1).astype(_F32)
        col16 = lax.broadcasted_iota(jnp.int32, (BLK, 16), 1).astype(_F32)
        idx_mat = jnp.full((BLK, 16), float(NQ), dtype=_F32)
        work = qsim
        for t in range(TOPK):
            mt = jnp.max(work, axis=1, keepdims=True)
            sel = jnp.min(
                jnp.where(work == mt, col, jnp.float32(NQ)), axis=1, keepdims=True
            )
            idx_mat = jnp.where(col16 == float(t), sel, idx_mat)
            work = jnp.where(col == sel, NEG_INF, work)
        idx_s[pl.ds(pl.multiple_of(j * BLK, BLK), BLK), :] = idx_mat
        idxt_s[:, pl.ds(pl.multiple_of(j * BLK, BLK), BLK)] = idx_mat.T

        @pl.when(j == 0)
        def _adapted_proto():
            proto = sup_s[:, 0 * D_OUT : 1 * D_OUT]
            for s in range(1, K_SHOT):
                proto = proto + sup_s[:, s * D_OUT : (s + 1) * D_OUT]
            proto = proto * (1.0 / K_SHOT)
            pn = jnp.sum(proto * proto, axis=1)
            pq = lax.dot_general(
                proto, qall, (((1,), (1,)), ((), ())), preferred_element_type=_F32
            )
            ps_t = 2.0 * pq - pn[:, None] - qnt

            row_n = lax.broadcasted_iota(jnp.int32, (N_WAY, NQ), 0).astype(_F32)
            m_col = jnp.max(ps_t, axis=0, keepdims=True)
            label = jnp.min(
                jnp.where(ps_t == m_col, row_n, jnp.float32(N_WAY)), axis=0,
                keepdims=True,
            )
            assign = row_n == label

            logq = jnp.where(assign, ps_t, NEG_INF)
            mx = jnp.maximum(jnp.max(logq, axis=1, keepdims=True), 0.0)
            e = jnp.exp(logq - mx)
            e_self = jnp.exp(-mx)
            den = jnp.sum(e, axis=1, keepdims=True) + e_self
            wq = e / den
            ap_s[...] = (
                lax.dot_general(
                    wq, qall, (((1,), (0,)), ((), ())), preferred_element_type=_F32
                )
                + (e_self / den) * proto
            )

    @pl.when(pid >= _NE + _NB)
    def _combine():
        j = pid - (_NE + _NB)
        qsim = qsim_s[pl.ds(pl.multiple_of(j * BLK, BLK), BLK), :]
        idx_mat = idx_s[pl.ds(pl.multiple_of(j * BLK, BLK), BLK), :]
        col = lax.broadcasted_iota(jnp.int32, (BLK, NQ), 1).astype(_F32)

        m_blk = jnp.zeros((BLK, NQ), dtype=jnp.bool_)
        mt_blk = jnp.zeros((BLK, NQ), dtype=jnp.bool_)
        row_glob = lax.broadcasted_iota(jnp.int32, (BLK, NQ), 0).astype(
            _F32
        ) + jnp.float32(BLK) * j.astype(_F32)
        for t in range(TOPK):
            m_blk = m_blk | (col == idx_mat[:, t : t + 1])
            mt_blk = mt_blk | (idxt_s[t : t + 1, :] == row_glob)
        mutual = m_blk & mt_blk

        q_log = jnp.where(mutual, qsim, NEG_INF)
        mq = jnp.max(q_log, axis=1, keepdims=True)
        e = jnp.exp(q_log - mq)
        q_w = e / jnp.sum(e, axis=1, keepdims=True)

        aq = lax.dot_general(
            q_w, qry_s[...], (((1,), (0,)), ((), ())), preferred_element_type=_F32
        )

        ap = ap_s[...]
        apn = jnp.sum(ap * ap, axis=1)
        aqn = jnp.sum(aq * aq, axis=1, keepdims=True)
        aqp = lax.dot_general(
            aq, ap, (((1,), (1,)), ((), ())), preferred_element_type=_F32
        )
        sim = 2.0 * aqp - aqn - apn[None, :]
        o_ref[...] = s_ref[0] * sim + s_ref[1]


def kernel(x, W, b, tao, n, k, q):
    residual = (
        (jnp.asarray(n) - N_WAY)
        + (jnp.asarray(k) - K_SHOT)
        + (jnp.asarray(q) - Q_QUERY)
    ).astype(x.dtype)
    scal = jnp.stack([tao.astype(_F32), residual.astype(_F32)])
    xblk = 25 * _CPB
    return pl.pallas_call(
        _body,
        grid=(_NE + 2 * _NB,),
        in_specs=[
            pl.BlockSpec((xblk, D_IN), lambda i: (jnp.minimum(i, _NE - 1), 0)),
            pl.BlockSpec((D_IN, D_OUT), lambda i: (0, 0)),
            pl.BlockSpec((1, D_OUT), lambda i: (0, 0)),
            pl.BlockSpec(memory_space=pltpu.SMEM),
        ],
        out_specs=pl.BlockSpec(
            (BLK, N_WAY), lambda i: (jnp.maximum(i - (_NE + _NB), 0), 0)
        ),
        out_shape=jax.ShapeDtypeStruct((NQ, N_WAY), _F32),
        scratch_shapes=[
            pltpu.VMEM((NQ, D_OUT), _F32),
            pltpu.VMEM((N_WAY, K_SHOT * D_OUT), _F32),
            pltpu.VMEM((NQ, NQ), _F32),
            pltpu.VMEM((NQ, 16), _F32),
            pltpu.VMEM((16, NQ), _F32),
            pltpu.VMEM((N_WAY, D_OUT), _F32),
        ],
    )(x, W, b.reshape(1, D_OUT), scal)
